# Initial kernel scaffold; baseline (speedup 1.0000x reference)
import jax
import jax.numpy as jnp
from jax.experimental import pallas as pl


def kernel(x, func_parameter, borders, inverse_chunk_lengths):
    raise NotImplementedError("write your pallas kernel here")


# speedup vs baseline: 31.6346x; 31.6346x over previous
import jax
import jax.numpy as jnp
from jax import lax
from jax.experimental import pallas as pl
from jax.experimental.pallas import tpu as pltpu
from jax.experimental.pallas import tpu_sc as plsc
import functools

G = 4096
IN = 64
OUT = 64
B = 16384
NC = 2
NS = 16
NW = NC * NS
BPW = B // NW
BLK = 128
NBLK = BPW // BLK
ROWS = G + 1


def _sc_body(x_hbm, fp_hbm, borders_hbm, invlen_hbm, out_hbm,
             borders_v, invlen_v, x_v, idxl_v, idxr_v, delta_v,
             bufl_v, bufr_v, acc_v, seml, semr):
    wid = lax.axis_index("s") * NC + lax.axis_index("c")
    base = wid * BPW

    pltpu.sync_copy(borders_hbm, borders_v)
    pltpu.sync_copy(invlen_hbm, invlen_v)

    zeros16 = jnp.zeros((16,), jnp.float32)

    def zero_body(b, c):
        for r in range(OUT // 16):
            acc_v[b, pl.ds(r * 16, 16)] = zeros16
        return c
    lax.fori_loop(0, BPW, zero_body, 0)

    def dim_body(i, c):
        pltpu.sync_copy(x_hbm.at[i, pl.ds(base, BPW)], x_v)

        def wgt_body(j, cc):
            xv = x_v[pl.ds(j * 16, 16)]
            e = jnp.exp(-jnp.abs(xv))
            cdf = jnp.where(xv > 0.0, 1.0 - 0.5 * e, 0.5 * e)
            idx = jnp.clip((cdf * float(G)).astype(jnp.int32), 0, G - 1)
            left = plsc.load_gather(borders_v, [idx])
            invl = plsc.load_gather(invlen_v, [idx])
            delta_v[pl.ds(j * 16, 16)] = (xv - left) * invl
            row = idx + i * ROWS
            idxl_v[pl.ds(j * 16, 16)] = row
            idxr_v[pl.ds(j * 16, 16)] = row + 1
            return cc
        lax.fori_loop(0, BPW // 16, wgt_body, 0)

        cps = []
        for blk in range(NBLK):
            cpl = pltpu.async_copy(
                fp_hbm.at[idxl_v.at[pl.ds(blk * BLK, BLK)]],
                bufl_v.at[pl.ds(blk * BLK, BLK)], seml)
            cpr = pltpu.async_copy(
                fp_hbm.at[idxr_v.at[pl.ds(blk * BLK, BLK)]],
                bufr_v.at[pl.ds(blk * BLK, BLK)], semr)
            cps.append((cpl, cpr))

        for blk in range(NBLK):
            cpl, cpr = cps[blk]
            cpl.wait()
            cpr.wait()

            def acc_body(j, cc):
                b = blk * BLK + j
                d = plsc.load_gather(delta_v, [jnp.full((16,), b, jnp.int32)])
                for r in range(OUT // 16):
                    L = bufl_v[b, pl.ds(r * 16, 16)]
                    R = bufr_v[b, pl.ds(r * 16, 16)]
                    plsc.addupdate(acc_v.at[b, pl.ds(r * 16, 16)],
                                   L + d * (R - L))
                return cc
            lax.fori_loop(0, BLK, acc_body, 0)
        return c

    lax.fori_loop(0, IN, dim_body, 0)

    pltpu.sync_copy(acc_v, out_hbm.at[pl.ds(base, BPW)])


@jax.jit
def _sc_call(x, fp2d, borders_pad, invlen):
    mesh = plsc.VectorSubcoreMesh(core_axis_name="c", subcore_axis_name="s",
                                  num_cores=NC, num_subcores=NS)
    f = pl.kernel(
        _sc_body,
        out_type=jax.ShapeDtypeStruct((B, OUT), jnp.float32),
        mesh=mesh,
        compiler_params=pltpu.CompilerParams(needs_layout_passes=False,
                                             use_tc_tiling_on_sc=False),
        scratch_types=[
            pltpu.VMEM((4112,), jnp.float32),
            pltpu.VMEM((G,), jnp.float32),
            pltpu.VMEM((BPW,), jnp.float32),
            pltpu.VMEM((BPW,), jnp.int32),
            pltpu.VMEM((BPW,), jnp.int32),
            pltpu.VMEM((BPW,), jnp.float32),
            pltpu.VMEM((BPW, OUT), jnp.float32),
            pltpu.VMEM((BPW, OUT), jnp.float32),
            pltpu.VMEM((BPW, OUT), jnp.float32),
            pltpu.SemaphoreType.DMA,
            pltpu.SemaphoreType.DMA,
        ],
    )
    return f(x, fp2d, borders_pad, invlen)


def kernel(x, func_parameter, borders, inverse_chunk_lengths):
    fp2d = jnp.transpose(func_parameter, (2, 0, 1)).reshape(IN * ROWS, OUT)
    borders_pad = jnp.pad(borders, (0, 4112 - ROWS))
    out_bt = _sc_call(x, fp2d, borders_pad, inverse_chunk_lengths)
    return out_bt.T


# speedup vs baseline: 39.0053x; 1.2330x over previous
import jax
import jax.numpy as jnp
from jax import lax
from jax.experimental import pallas as pl
from jax.experimental.pallas import tpu as pltpu
from jax.experimental.pallas import tpu_sc as plsc

G = 4096
IN = 64
OUT = 64
B = 16384
NC = 2
NS = 16
NW = NC * NS
BPW = B // NW
BLK = 128
NBLK = BPW // BLK
ROWS = G + 1


def _sc_body(x_hbm, fp_hbm, borders_hbm, invlen_hbm, out_hbm,
             borders_v, invlen_v, x_v, idxl_v, idxr_v, delta_v,
             bufl_v, bufr_v, acc_v, semx, seml, semr):
    wid = lax.axis_index("s") * NC + lax.axis_index("c")
    base = wid * BPW

    pltpu.sync_copy(borders_hbm, borders_v)
    pltpu.sync_copy(invlen_hbm, invlen_v)

    zeros16 = jnp.zeros((16,), jnp.float32)

    @plsc.parallel_loop(0, BPW, unroll=4)
    def _(b):
        for r in range(OUT // 16):
            acc_v[b, pl.ds(r * 16, 16)] = zeros16

    def compute_weights(i1, par):
        def wgt_body(j, cc):
            xv = x_v[par, pl.ds(j * 16, 16)]
            e = jnp.exp(-jnp.abs(xv))
            cdf = jnp.where(xv > 0.0, 1.0 - 0.5 * e, 0.5 * e)
            idx = jnp.clip((cdf * float(G)).astype(jnp.int32), 0, G - 1)
            left = plsc.load_gather(borders_v, [idx])
            invl = plsc.load_gather(invlen_v, [idx])
            delta_v[par, pl.ds(j * 16, 16)] = (xv - left) * invl
            row = idx + i1 * ROWS
            idxl_v[par, pl.ds(j * 16, 16)] = row
            idxr_v[par, pl.ds(j * 16, 16)] = row + 1
            return cc
        lax.fori_loop(0, BPW // 16, wgt_body, 0)

    def fire_block(par, blk):
        pltpu.async_copy(
            fp_hbm.at[idxl_v.at[par, pl.ds(blk * BLK, BLK)]],
            bufl_v.at[pl.ds(blk * BLK, BLK)], seml)
        pltpu.async_copy(
            fp_hbm.at[idxr_v.at[par, pl.ds(blk * BLK, BLK)]],
            bufr_v.at[pl.ds(blk * BLK, BLK)], semr)

    def wait_block(par, blk):
        pltpu.make_async_copy(
            fp_hbm.at[idxl_v.at[par, pl.ds(blk * BLK, BLK)]],
            bufl_v.at[pl.ds(blk * BLK, BLK)], seml).wait()
        pltpu.make_async_copy(
            fp_hbm.at[idxr_v.at[par, pl.ds(blk * BLK, BLK)]],
            bufr_v.at[pl.ds(blk * BLK, BLK)], semr).wait()

    pltpu.sync_copy(x_hbm.at[0, pl.ds(base, BPW)], x_v.at[0])
    pltpu.async_copy(x_hbm.at[1, pl.ds(base, BPW)], x_v.at[1], semx)
    compute_weights(0, 0)
    for blk in range(NBLK):
        fire_block(0, blk)

    def dim_body(i, c):
        par = lax.rem(i, 2)
        parn = 1 - par

        @pl.when(i < IN - 1)
        def _():
            pltpu.make_async_copy(
                x_hbm.at[i + 1, pl.ds(base, BPW)], x_v.at[parn], semx).wait()

            @pl.when(i < IN - 2)
            def _():
                pltpu.async_copy(
                    x_hbm.at[i + 2, pl.ds(base, BPW)], x_v.at[par], semx)

            compute_weights(i + 1, parn)

        for blk in range(NBLK):
            wait_block(par, blk)

            @plsc.parallel_loop(0, BLK, unroll=4)
            def _(j):
                b = blk * BLK + j
                d = plsc.load_gather(
                    delta_v.at[par], [jnp.full((16,), b, jnp.int32)])
                for r in range(OUT // 16):
                    L = bufl_v[b, pl.ds(r * 16, 16)]
                    R = bufr_v[b, pl.ds(r * 16, 16)]
                    plsc.addupdate(acc_v.at[b, pl.ds(r * 16, 16)],
                                   L + d * (R - L))

            @pl.when(i < IN - 1)
            def _():
                fire_block(parn, blk)

        return c

    lax.fori_loop(0, IN, dim_body, 0)

    pltpu.sync_copy(acc_v, out_hbm.at[pl.ds(base, BPW)])


@jax.jit
def _sc_call(x, fp2d, borders_pad, invlen):
    mesh = plsc.VectorSubcoreMesh(core_axis_name="c", subcore_axis_name="s",
                                  num_cores=NC, num_subcores=NS)
    f = pl.kernel(
        _sc_body,
        out_type=jax.ShapeDtypeStruct((B, OUT), jnp.float32),
        mesh=mesh,
        compiler_params=pltpu.CompilerParams(needs_layout_passes=False,
                                             use_tc_tiling_on_sc=False),
        scratch_types=[
            pltpu.VMEM((4112,), jnp.float32),
            pltpu.VMEM((G,), jnp.float32),
            pltpu.VMEM((2, BPW), jnp.float32),
            pltpu.VMEM((2, BPW), jnp.int32),
            pltpu.VMEM((2, BPW), jnp.int32),
            pltpu.VMEM((2, BPW), jnp.float32),
            pltpu.VMEM((BPW, OUT), jnp.float32),
            pltpu.VMEM((BPW, OUT), jnp.float32),
            pltpu.VMEM((BPW, OUT), jnp.float32),
            pltpu.SemaphoreType.DMA,
            pltpu.SemaphoreType.DMA,
            pltpu.SemaphoreType.DMA,
        ],
    )
    return f(x, fp2d, borders_pad, invlen)


def kernel(x, func_parameter, borders, inverse_chunk_lengths):
    fp2d = jnp.transpose(func_parameter, (2, 0, 1)).reshape(IN * ROWS, OUT)
    borders_pad = jnp.pad(borders, (0, 4112 - ROWS))
    out_bt = _sc_call(x, fp2d, borders_pad, inverse_chunk_lengths)
    return out_bt.T
